# baseline (speedup 1.0000x reference)
import functools

import jax
import jax.numpy as jnp
from jax import lax
from jax.experimental import pallas as pl
from jax.experimental.pallas import tpu as pltpu
from jax.experimental.pallas import tpu_sc as plsc

N = 10000
D = 128
E = 320000

NC = 2
NS = 16
NW = NC * NS
CH = 128
NCHUNK = -(-E // (NW * CH))
NCH0 = 132
NCH1 = 2 * NCHUNK - NCH0
E_PAD = NS * (NCH0 + NCH1) * CH
AGG_ROWS = 10112
ZROWS = AGG_ROWS // NS


def _sc_segment_sum(h, edges3):
    mesh = plsc.VectorSubcoreMesh(core_axis_name="c", subcore_axis_name="s")

    @functools.partial(
        pl.kernel,
        mesh=mesh,
        out_type=jax.ShapeDtypeStruct((NC, AGG_ROWS, D), jnp.float32),
        scratch_types=[
            pltpu.VMEM((3, 2, CH), jnp.int32),
            pltpu.VMEM((2, CH, D), jnp.float32),
            pltpu.VMEM_SHARED((AGG_ROWS, D), jnp.float32),
        ] + [pltpu.SemaphoreType.DMA] * 2,
    )
    def sc_kernel(h_hbm, e_hbm, out_hbm, idx_v, rows_v, agg_sp, *sems):
        gsem, isem = sems
        c = lax.axis_index("c")
        s = lax.axis_index("s")
        nch = jnp.where(c == 0, NCH0, NCH1)
        base = jnp.where(c == 0, s * NCH0, NS * NCH0 + s * NCH1)
        pltpu.sync_copy(e_hbm.at[base], idx_v.at[0])
        pltpu.async_copy(e_hbm.at[base + 1], idx_v.at[1], isem)
        pltpu.async_copy(h_hbm.at[idx_v.at[0, 0]], rows_v.at[0], gsem)

        zvec = jnp.zeros((16,), jnp.float32)

        def zbody(i, carry):
            for k in range(D // 16):
                rows_v[1, i, pl.ds(k * 16, 16)] = zvec
            return carry

        lax.fori_loop(0, CH, zbody, 0)
        full, rem = divmod(ZROWS, CH)
        for t in range(full):
            pltpu.sync_copy(rows_v.at[1],
                            agg_sp.at[pl.ds(s * ZROWS + t * CH, CH)])
        if rem:
            pltpu.sync_copy(rows_v.at[1, pl.ds(0, rem)],
                            agg_sp.at[pl.ds(s * ZROWS + full * CH, rem)])
        plsc.subcore_barrier()

        def body(j, carry):
            p = lax.rem(j, 2)
            sj = lax.rem(j, 3)
            sn = lax.rem(j + 1, 3)
            sp = lax.rem(j + 2, 3)

            pltpu.make_async_copy(h_hbm.at[idx_v.at[sj, 0]], rows_v.at[p],
                                  gsem).wait()

            @pl.when(j + 1 < nch)
            def _():
                pltpu.make_async_copy(e_hbm.at[base + j + 1], idx_v.at[sn],
                                      isem).wait()
                pltpu.async_copy(h_hbm.at[idx_v.at[sn, 0]], rows_v.at[1 - p],
                                 gsem)

            @pl.when(j + 2 < nch)
            def _():
                pltpu.async_copy(e_hbm.at[base + j + 2], idx_v.at[sp], isem)

            pltpu.sync_copy(rows_v.at[p], agg_sp.at[idx_v.at[sj, 1]], add=True)
            return carry

        lax.fori_loop(0, nch, body, 0)
        plsc.subcore_barrier()
        pltpu.sync_copy(agg_sp.at[pl.ds(s * ZROWS, ZROWS)],
                        out_hbm.at[c, pl.ds(s * ZROWS, ZROWS)])

    return sc_kernel(h, edges3)


def _dense_body(p_ref, h_ref, c_ref, wr_ref, wt_ref, b_ref, hn_ref, cn_ref):
    agg = p_ref[0] + p_ref[1]
    g = (jnp.dot(agg, wr_ref[...], preferred_element_type=jnp.float32)
         + jnp.dot(h_ref[...], wt_ref[...], preferred_element_type=jnp.float32)
         + b_ref[...])
    z = jnp.tanh(g[:, 0:D])
    i = jax.nn.sigmoid(g[:, D:2 * D])
    f = jax.nn.sigmoid(g[:, 2 * D:3 * D])
    o = jax.nn.sigmoid(g[:, 3 * D:4 * D])
    cn = f * c_ref[...] + i * z
    cn_ref[...] = cn
    hn_ref[...] = o * jnp.tanh(cn)


def _dense(partials, h, c, w_rel, w_root, b):
    blk = 1000
    grid = N // blk
    return pl.pallas_call(
        _dense_body,
        grid=(grid,),
        in_specs=[
            pl.BlockSpec((NC, blk, D), lambda n: (0, n, 0)),
            pl.BlockSpec((blk, D), lambda n: (n, 0)),
            pl.BlockSpec((blk, D), lambda n: (n, 0)),
            pl.BlockSpec((D, 4 * D), lambda n: (0, 0)),
            pl.BlockSpec((D, 4 * D), lambda n: (0, 0)),
            pl.BlockSpec((1, 4 * D), lambda n: (0, 0)),
        ],
        out_specs=[
            pl.BlockSpec((blk, D), lambda n: (n, 0)),
            pl.BlockSpec((blk, D), lambda n: (n, 0)),
        ],
        out_shape=[
            jax.ShapeDtypeStruct((N, D), jnp.float32),
            jax.ShapeDtypeStruct((N, D), jnp.float32),
        ],
    )(partials, h, c, w_rel, w_root, b)


def kernel(h, c, row, col, batch, Wz_root, bz, Wz_rel, Wi_root, bi, Wi_rel,
           Wf_root, bf, Wf_rel, Wo_root, bo, Wo_rel):
    pad = E_PAD - E
    row_p = jnp.concatenate([row, jnp.zeros((pad,), jnp.int32)])
    col_p = jnp.concatenate([col, jnp.full((pad,), N, jnp.int32)])
    edges3 = jnp.concatenate(
        [row_p.reshape(NW * NCHUNK, 1, CH), col_p.reshape(NW * NCHUNK, 1, CH)],
        axis=1)

    w_rel = jnp.concatenate(
        [Wz_rel.T, Wi_rel.T, Wf_rel.T, Wo_rel.T], axis=1)
    w_root = jnp.concatenate(
        [Wz_root.T, Wi_root.T, Wf_root.T, Wo_root.T], axis=1)
    b = jnp.concatenate([bz, bi, bf, bo]).reshape(1, 4 * D)

    partials = _sc_segment_sum(h, edges3)

    h_new, c_new = _dense(partials, h, c, w_rel, w_root, b)
    return (h_new, c_new)


# speedup vs baseline: 1.0135x; 1.0135x over previous
import functools

import jax
import jax.numpy as jnp
from jax import lax
from jax.experimental import pallas as pl
from jax.experimental.pallas import tpu as pltpu
from jax.experimental.pallas import tpu_sc as plsc

N = 10000
D = 128
E = 320000

NC = 2
NS = 16
NW = NC * NS
CH = 128
NCHUNK = -(-E // (NW * CH))
NCH0 = 130
NCH1 = 2 * NCHUNK - NCH0
E_PAD = NS * (NCH0 + NCH1) * CH
AGG_ROWS = 10112
ZROWS = AGG_ROWS // NS


def _sc_segment_sum(h, edges3):
    mesh = plsc.VectorSubcoreMesh(core_axis_name="c", subcore_axis_name="s")

    @functools.partial(
        pl.kernel,
        mesh=mesh,
        out_type=jax.ShapeDtypeStruct((NC, AGG_ROWS, D), jnp.float32),
        scratch_types=[
            pltpu.VMEM((3, 2, CH), jnp.int32),
            pltpu.VMEM((2, CH, D), jnp.float32),
            pltpu.VMEM_SHARED((AGG_ROWS, D), jnp.float32),
        ] + [pltpu.SemaphoreType.DMA] * 2,
    )
    def sc_kernel(h_hbm, e_hbm, out_hbm, idx_v, rows_v, agg_sp, *sems):
        gsem, isem = sems
        c = lax.axis_index("c")
        s = lax.axis_index("s")
        nch = jnp.where(c == 0, NCH0, NCH1)
        base = jnp.where(c == 0, s * NCH0, NS * NCH0 + s * NCH1)
        pltpu.sync_copy(e_hbm.at[base], idx_v.at[0])
        pltpu.async_copy(e_hbm.at[base + 1], idx_v.at[1], isem)
        pltpu.async_copy(h_hbm.at[idx_v.at[0, 0]], rows_v.at[0], gsem)

        zvec = jnp.zeros((16,), jnp.float32)

        def zbody(i, carry):
            for k in range(D // 16):
                rows_v[1, i, pl.ds(k * 16, 16)] = zvec
            return carry

        lax.fori_loop(0, CH, zbody, 0)
        full, rem = divmod(ZROWS, CH)
        for t in range(full):
            pltpu.sync_copy(rows_v.at[1],
                            agg_sp.at[pl.ds(s * ZROWS + t * CH, CH)])
        if rem:
            pltpu.sync_copy(rows_v.at[1, pl.ds(0, rem)],
                            agg_sp.at[pl.ds(s * ZROWS + full * CH, rem)])
        plsc.subcore_barrier()

        def body(j, carry):
            p = lax.rem(j, 2)
            sj = lax.rem(j, 3)
            sn = lax.rem(j + 1, 3)
            sp = lax.rem(j + 2, 3)

            pltpu.make_async_copy(h_hbm.at[idx_v.at[sj, 0]], rows_v.at[p],
                                  gsem).wait()

            @pl.when(j + 1 < nch)
            def _():
                pltpu.make_async_copy(e_hbm.at[base + j + 1], idx_v.at[sn],
                                      isem).wait()
                pltpu.async_copy(h_hbm.at[idx_v.at[sn, 0]], rows_v.at[1 - p],
                                 gsem)

            @pl.when(j + 2 < nch)
            def _():
                pltpu.async_copy(e_hbm.at[base + j + 2], idx_v.at[sp], isem)

            pltpu.sync_copy(rows_v.at[p], agg_sp.at[idx_v.at[sj, 1]], add=True)
            return carry

        lax.fori_loop(0, nch, body, 0)
        plsc.subcore_barrier()
        pltpu.sync_copy(agg_sp.at[pl.ds(s * ZROWS, ZROWS)],
                        out_hbm.at[c, pl.ds(s * ZROWS, ZROWS)])

    return sc_kernel(h, edges3)


def _dense_body(p_ref, h_ref, c_ref, wr_ref, wt_ref, b_ref, hn_ref, cn_ref):
    agg = p_ref[0] + p_ref[1]
    g = (jnp.dot(agg, wr_ref[...], preferred_element_type=jnp.float32)
         + jnp.dot(h_ref[...], wt_ref[...], preferred_element_type=jnp.float32)
         + b_ref[...])
    z = jnp.tanh(g[:, 0:D])
    i = jax.nn.sigmoid(g[:, D:2 * D])
    f = jax.nn.sigmoid(g[:, 2 * D:3 * D])
    o = jax.nn.sigmoid(g[:, 3 * D:4 * D])
    cn = f * c_ref[...] + i * z
    cn_ref[...] = cn
    hn_ref[...] = o * jnp.tanh(cn)


def _dense(partials, h, c, w_rel, w_root, b):
    blk = 1000
    grid = N // blk
    return pl.pallas_call(
        _dense_body,
        grid=(grid,),
        in_specs=[
            pl.BlockSpec((NC, blk, D), lambda n: (0, n, 0)),
            pl.BlockSpec((blk, D), lambda n: (n, 0)),
            pl.BlockSpec((blk, D), lambda n: (n, 0)),
            pl.BlockSpec((D, 4 * D), lambda n: (0, 0)),
            pl.BlockSpec((D, 4 * D), lambda n: (0, 0)),
            pl.BlockSpec((1, 4 * D), lambda n: (0, 0)),
        ],
        out_specs=[
            pl.BlockSpec((blk, D), lambda n: (n, 0)),
            pl.BlockSpec((blk, D), lambda n: (n, 0)),
        ],
        out_shape=[
            jax.ShapeDtypeStruct((N, D), jnp.float32),
            jax.ShapeDtypeStruct((N, D), jnp.float32),
        ],
    )(partials, h, c, w_rel, w_root, b)


def kernel(h, c, row, col, batch, Wz_root, bz, Wz_rel, Wi_root, bi, Wi_rel,
           Wf_root, bf, Wf_rel, Wo_root, bo, Wo_rel):
    pad = E_PAD - E
    row_p = jnp.concatenate([row, jnp.zeros((pad,), jnp.int32)])
    col_p = jnp.concatenate([col, jnp.full((pad,), N, jnp.int32)])
    edges3 = jnp.concatenate(
        [row_p.reshape(NW * NCHUNK, 1, CH), col_p.reshape(NW * NCHUNK, 1, CH)],
        axis=1)

    w_rel = jnp.concatenate(
        [Wz_rel.T, Wi_rel.T, Wf_rel.T, Wo_rel.T], axis=1)
    w_root = jnp.concatenate(
        [Wz_root.T, Wi_root.T, Wf_root.T, Wo_root.T], axis=1)
    b = jnp.concatenate([bz, bi, bf, bo]).reshape(1, 4 * D)

    partials = _sc_segment_sum(h, edges3)

    h_new, c_new = _dense(partials, h, c, w_rel, w_root, b)
    return (h_new, c_new)
